# Initial kernel scaffold; baseline (speedup 1.0000x reference)
import jax
import jax.numpy as jnp
from jax.experimental import pallas as pl


def kernel(positions, grid_positions, cached_metrics):
    raise NotImplementedError("write your pallas kernel here")


# speedup vs baseline: 2.5151x; 2.5151x over previous
import functools

import jax
import jax.numpy as jnp
from jax import lax
from jax.experimental import pallas as pl
from jax.experimental.pallas import tpu as pltpu
from jax.experimental.pallas import tpu_sc as plsc

Q, K, D = 4096, 16384, 64
DD = D * D
KN = 4
TQ = 256
TK = 2048
NKC = K // TK

NC, NS, L = 2, 16, 16
NW = NC * NS
QPW = Q // NW
NP = QPW // 2


def _topk_body(q_ref, g_ref, idx_ref, w_ref):
    q = q_ref[...]
    qsq = jnp.sum(q * q, axis=1, keepdims=True)

    def chunk_step(kc, carry):
        bv, bi = carry
        g = g_ref[pl.ds(kc * TK, TK), :]
        gsq = jnp.sum(g * g, axis=1)[None, :]
        cross = lax.dot_general(q, g, (((1,), (1,)), ((), ())),
                                preferred_element_type=jnp.float32)
        d2 = jnp.maximum(qsq + gsq - 2.0 * cross, 0.0)
        lane = lax.broadcasted_iota(jnp.int32, (TQ, TK), 1)
        cur = d2
        cv, ci = [], []
        for _ in range(KN):
            m = jnp.min(cur, axis=1, keepdims=True)
            am = jnp.min(jnp.where(cur <= m, lane, K), axis=1, keepdims=True)
            cv.append(m)
            ci.append(am + kc * TK)
            cur = jnp.where(lane == am, jnp.float32(jnp.inf), cur)
        av = jnp.concatenate([bv] + cv, axis=1)
        ai = jnp.concatenate([bi] + ci, axis=1)
        lane8 = lax.broadcasted_iota(jnp.int32, (TQ, 2 * KN), 1)
        cur = av
        ov, oi = [], []
        for _ in range(KN):
            m = jnp.min(cur, axis=1, keepdims=True)
            am = jnp.min(jnp.where(cur <= m, lane8, 2 * KN), axis=1,
                         keepdims=True)
            ov.append(m)
            oi.append(jnp.sum(jnp.where(lane8 == am, ai, 0), axis=1,
                              keepdims=True))
            cur = jnp.where(lane8 == am, jnp.float32(jnp.inf), cur)
        return jnp.concatenate(ov, axis=1), jnp.concatenate(oi, axis=1)

    bv0 = jnp.full((TQ, KN), jnp.inf, jnp.float32)
    bi0 = jnp.zeros((TQ, KN), jnp.int32)
    bv, bi = lax.fori_loop(0, NKC, chunk_step, (bv0, bi0))
    dist = jnp.sqrt(bv)
    wgt = 1.0 / (dist + 1e-6)
    wgt = wgt / jnp.sum(wgt, axis=1, keepdims=True)
    idx_ref[...] = bi
    w_ref[...] = wgt


def _topk(positions, grid_positions, interpret=False):
    return pl.pallas_call(
        _topk_body,
        grid=(Q // TQ,),
        in_specs=[
            pl.BlockSpec((TQ, D), lambda i: (i, 0)),
            pl.BlockSpec((K, D), lambda i: (0, 0)),
        ],
        out_specs=[
            pl.BlockSpec((TQ, KN), lambda i: (i, 0)),
            pl.BlockSpec((TQ, KN), lambda i: (i, 0)),
        ],
        out_shape=[
            jax.ShapeDtypeStruct((Q, KN), jnp.int32),
            jax.ShapeDtypeStruct((Q, KN), jnp.float32),
        ],
        interpret=interpret,
    )(positions, grid_positions)


def _blend_body(tbl_hbm, idx_hbm, wb_hbm, out_hbm,
                idx_v, wb_v, rows_v, out_v, gsem):
    wid = lax.axis_index("s") * NC + lax.axis_index("c")
    qbase = wid * QPW
    pltpu.sync_copy(idx_hbm.at[pl.ds(qbase * KN, QPW * KN)], idx_v)
    pltpu.sync_copy(wb_hbm.at[pl.ds(qbase, QPW)], wb_v)

    def pair_step(p, _):
        pltpu.async_copy(tbl_hbm.at[idx_v.at[pl.ds(p * 2 * KN, 2 * KN)]],
                         rows_v, gsem).wait()
        w00 = wb_v[2 * p, 0]
        w01 = wb_v[2 * p, 1]
        w02 = wb_v[2 * p, 2]
        w03 = wb_v[2 * p, 3]
        w10 = wb_v[2 * p + 1, 0]
        w11 = wb_v[2 * p + 1, 1]
        w12 = wb_v[2 * p + 1, 2]
        w13 = wb_v[2 * p + 1, 3]

        def col_step(c, _):
            sl = pl.ds(c * L, L)
            acc0 = w00 * rows_v[0, sl]
            acc0 = acc0 + w01 * rows_v[1, sl]
            acc0 = acc0 + w02 * rows_v[2, sl]
            acc0 = acc0 + w03 * rows_v[3, sl]
            out_v[0, sl] = acc0
            acc1 = w10 * rows_v[4, sl]
            acc1 = acc1 + w11 * rows_v[5, sl]
            acc1 = acc1 + w12 * rows_v[6, sl]
            acc1 = acc1 + w13 * rows_v[7, sl]
            out_v[1, sl] = acc1
            return 0

        lax.fori_loop(0, DD // L, col_step, 0)
        pltpu.sync_copy(out_v, out_hbm.at[pl.ds(qbase + 2 * p, 2)])
        return 0

    lax.fori_loop(0, NP, pair_step, 0)


def _blend(tbl, idx_flat, wb):
    mesh = plsc.VectorSubcoreMesh(core_axis_name="c", subcore_axis_name="s",
                                  num_cores=NC, num_subcores=NS)
    f = pl.kernel(
        _blend_body,
        out_type=jax.ShapeDtypeStruct((Q, DD), jnp.float32),
        mesh=mesh,
        scratch_types=[
            pltpu.VMEM((QPW * KN,), jnp.int32),
            pltpu.VMEM((QPW, KN, L), jnp.float32),
            pltpu.VMEM((2 * KN, DD), jnp.float32),
            pltpu.VMEM((2, DD), jnp.float32),
            pltpu.SemaphoreType.DMA,
        ],
    )
    return f(tbl, idx_flat, wb)


def kernel(positions, grid_positions, cached_metrics):
    idx, w = _topk(positions, grid_positions)
    wb = jnp.broadcast_to(w[:, :, None], (Q, KN, L))
    tbl = cached_metrics.reshape(K, DD)
    out = _blend(tbl, idx.reshape(-1), wb)
    return out.reshape(Q, D, D)
